# baseline (speedup 1.0000x reference)
import jax
import jax.numpy as jnp
from jax.experimental import pallas as pl
from jax.experimental.pallas import tpu as pltpu

LANE = 128
TN = 128
TE = 2048
VMEM_LIMIT = 56 * 1024 * 1024
_SHIFT = 13
_MASK = (1 << _SHIFT) - 1


def kernel(nfeats, efeats, src, dst,
           l0_Wm_n, l0_Wm_e, l0_b_msg, l0_Wa_s, l0_Wa_n, l0_b_apply,
           l1_Wm_n, l1_Wm_e, l1_b_msg, l1_Wa_s, l1_Wa_n, l1_b_apply):
    N = nfeats.shape[0]
    E = efeats.shape[0]
    h0 = nfeats.reshape(N, LANE).astype(jnp.float32)
    ef = efeats.reshape(E, LANE).astype(jnp.float32)
    src32 = src.astype(jnp.int32)
    dst32 = dst.astype(jnp.int32)

    NB = N // TN
    NTILES = E // TE
    P1 = NTILES + NB
    P2 = P1 + NTILES
    GRID = P2 + NB
    blk_shift = TN.bit_length() - 1

    iota_e = jnp.arange(E, dtype=jnp.int32)
    dst_s, src_s, order = jax.lax.sort((dst32, src32, iota_e), num_keys=1)
    packed = ((src_s << _SHIFT) | dst_s).reshape(1, E)
    eid = order.reshape(1, E)

    def agg_tile(pk_smem, eid_smem, pk_vmem, hsrc_ref, ef_ref,
                 slabh_ref, slabe_ref, acch_ref, acce_ref, accd_ref):
        for mi in range(TE):
            slabh_ref[mi, :] = hsrc_ref[pk_smem[0, mi] >> _SHIFT, :]
            if ef_ref is not None:
                slabe_ref[mi, :] = ef_ref[eid_smem[0, mi], :]
        d = pk_vmem[...] & _MASK
        b_lo = (pk_smem[0, 0] & _MASK) >> blk_shift
        b_hi = (pk_smem[0, TE - 1] & _MASK) >> blk_shift
        rows = jax.lax.broadcasted_iota(jnp.int32, (TN, TE), 0)

        def body(b, carry):
            sl = pl.ds(b * TN, TN)
            onehot = (rows == (d - b * TN)).astype(jnp.float32)
            acch_ref[sl, :] += jnp.dot(
                onehot, slabh_ref[...], preferred_element_type=jnp.float32)
            if ef_ref is not None:
                acce_ref[sl, :] += jnp.dot(
                    onehot, slabe_ref[...], preferred_element_type=jnp.float32)
                accd_ref[sl, :] += jnp.sum(onehot, axis=1, keepdims=True)
            return carry

        jax.lax.fori_loop(b_lo, b_hi + 1, body, 0)

    def apply_block(acc_h, acc_e, h_self, invd, wmn_ref, wme_ref, bm_ref,
                    was_ref, wan_ref, ba_ref):
        hn = (jnp.dot(acc_h, wmn_ref[...], preferred_element_type=jnp.float32)
              + jnp.dot(acc_e, wme_ref[...], preferred_element_type=jnp.float32)
              ) * invd
        hn = hn + jnp.where(invd > 0, 1.0, 0.0) * bm_ref[...]
        z = (jnp.dot(h_self, was_ref[...], preferred_element_type=jnp.float32)
             + jnp.dot(hn, wan_ref[...], preferred_element_type=jnp.float32)
             + ba_ref[...])
        return jnp.maximum(z, 0.0)

    def mega_kernel(pk_smem, eid_smem, pk_vmem, h0_ref, ef_ref,
                    wmn0, wme0, bm0, was0, wan0, ba0,
                    wmn1, wme1, bm1, was1, wan1, ba1,
                    out_ref, slabh_ref, slabe_ref,
                    acch_ref, acce_ref, accd_ref, h1_ref):
        t = pl.program_id(0)

        @pl.when(t == 0)
        def _():
            acch_ref[...] = jnp.zeros_like(acch_ref)
            acce_ref[...] = jnp.zeros_like(acce_ref)
            accd_ref[...] = jnp.zeros_like(accd_ref)

        @pl.when(t < NTILES)
        def _():
            agg_tile(pk_smem, eid_smem, pk_vmem, h0_ref, ef_ref,
                     slabh_ref, slabe_ref, acch_ref, acce_ref, accd_ref)

        @pl.when(jnp.logical_and(t >= NTILES, t < P1))
        def _():
            b = t - NTILES
            sl = pl.ds(b * TN, TN)
            cnt = accd_ref[sl, :]
            invd = jnp.where(cnt > 0, 1.0 / cnt, 0.0)
            h1_ref[sl, :] = apply_block(acch_ref[sl, :], acce_ref[sl, :],
                                        h0_ref[sl, :], invd,
                                        wmn0, wme0, bm0, was0, wan0, ba0)

        @pl.when(t == P1)
        def _():
            acch_ref[...] = jnp.zeros_like(acch_ref)

        @pl.when(jnp.logical_and(t >= P1, t < P2))
        def _():
            agg_tile(pk_smem, eid_smem, pk_vmem, h1_ref, None,
                     slabh_ref, None, acch_ref, None, None)

        @pl.when(t >= P2)
        def _():
            b = t - P2
            sl = pl.ds(b * TN, TN)
            cnt = accd_ref[sl, :]
            invd = jnp.where(cnt > 0, 1.0 / cnt, 0.0)
            out_ref[...] = apply_block(acch_ref[sl, :], acce_ref[sl, :],
                                       h1_ref[sl, :], invd,
                                       wmn1, wme1, bm1, was1, wan1, ba1)

    def tile_map(t):
        u = jnp.where(t < P1, t, t - P1)
        return (0, jnp.clip(u, 0, NTILES - 1))

    def out_map(t):
        return (jnp.maximum(t - P2, 0), 0)

    rspec = lambda shape: pl.BlockSpec(shape, lambda t: (0, 0))
    wspecs = [rspec((LANE, LANE)), rspec((LANE, LANE)), rspec((1, LANE)),
              rspec((LANE, LANE)), rspec((LANE, LANE)), rspec((1, LANE))]

    out1 = pl.pallas_call(
        mega_kernel,
        out_shape=jax.ShapeDtypeStruct((N, LANE), jnp.float32),
        grid_spec=pltpu.PrefetchScalarGridSpec(
            num_scalar_prefetch=0,
            grid=(GRID,),
            in_specs=[
                pl.BlockSpec((1, TE), tile_map, memory_space=pltpu.SMEM),
                pl.BlockSpec((1, TE), tile_map, memory_space=pltpu.SMEM),
                pl.BlockSpec((1, TE), tile_map),
                rspec((N, LANE)),
                rspec((E, LANE)),
                *wspecs, *wspecs,
            ],
            out_specs=pl.BlockSpec((TN, LANE), out_map),
            scratch_shapes=[pltpu.VMEM((TE, LANE), jnp.float32),
                            pltpu.VMEM((TE, LANE), jnp.float32),
                            pltpu.VMEM((N, LANE), jnp.float32),
                            pltpu.VMEM((N, LANE), jnp.float32),
                            pltpu.VMEM((N, 1), jnp.float32),
                            pltpu.VMEM((N, LANE), jnp.float32)],
        ),
        compiler_params=pltpu.CompilerParams(
            dimension_semantics=("arbitrary",),
            vmem_limit_bytes=VMEM_LIMIT,
        ),
    )(packed, eid, packed, h0, ef,
      l0_Wm_n, l0_Wm_e, l0_b_msg, l0_Wa_s, l0_Wa_n, l0_b_apply,
      l1_Wm_n, l1_Wm_e, l1_b_msg, l1_Wa_s, l1_Wa_n, l1_b_apply)

    return out1


# speedup vs baseline: 1.1275x; 1.1275x over previous
import jax
import jax.numpy as jnp
from jax.experimental import pallas as pl
from jax.experimental.pallas import tpu as pltpu

LANE = 128
TN = 128
TE = 1024
VMEM_LIMIT = 56 * 1024 * 1024
_SHIFT = 13
_MASK = (1 << _SHIFT) - 1


def kernel(nfeats, efeats, src, dst,
           l0_Wm_n, l0_Wm_e, l0_b_msg, l0_Wa_s, l0_Wa_n, l0_b_apply,
           l1_Wm_n, l1_Wm_e, l1_b_msg, l1_Wa_s, l1_Wa_n, l1_b_apply):
    N = nfeats.shape[0]
    E = efeats.shape[0]
    h0 = nfeats.reshape(N, LANE).astype(jnp.float32)
    ef = efeats.reshape(E, LANE).astype(jnp.float32)
    src32 = src.astype(jnp.int32)
    dst32 = dst.astype(jnp.int32)

    NB = N // TN
    NTILES = E // TE
    P1 = NTILES + NB
    P2 = P1 + NTILES
    GRID = P2 + NB
    blk_shift = TN.bit_length() - 1

    iota_e = jnp.arange(E, dtype=jnp.int32)
    dst_s, src_s, order = jax.lax.sort((dst32, src32, iota_e), num_keys=1)
    srcs = src_s.reshape(1, E)
    dsts = dst_s.reshape(1, E)
    eid = order.reshape(1, E)

    def agg_tile(src_smem, eid_smem, dst_smem, dst_vmem, hsrc_ref, ef_ref,
                 slabh_ref, slabe_ref, acch_ref, acce_ref, accd_ref):
        for mi in range(TE):
            slabh_ref[mi, :] = hsrc_ref[src_smem[0, mi], :]
            if ef_ref is not None:
                slabe_ref[mi, :] = ef_ref[eid_smem[0, mi], :]
        d = dst_vmem[...]
        b_lo = dst_smem[0, 0] >> blk_shift
        b_hi = dst_smem[0, TE - 1] >> blk_shift
        rows = jax.lax.broadcasted_iota(jnp.int32, (TN, TE), 0)

        def body(b, carry):
            sl = pl.ds(b * TN, TN)
            onehot = (rows == (d - b * TN)).astype(jnp.float32)
            acch_ref[sl, :] += jnp.dot(
                onehot, slabh_ref[...], preferred_element_type=jnp.float32)
            if ef_ref is not None:
                acce_ref[sl, :] += jnp.dot(
                    onehot, slabe_ref[...], preferred_element_type=jnp.float32)
                accd_ref[sl, :] += jnp.sum(onehot, axis=1, keepdims=True)
            return carry

        jax.lax.fori_loop(b_lo, b_hi + 1, body, 0)

    def apply_block(acc_h, acc_e, h_self, invd, wmn_ref, wme_ref, bm_ref,
                    was_ref, wan_ref, ba_ref):
        hn = (jnp.dot(acc_h, wmn_ref[...], preferred_element_type=jnp.float32)
              + jnp.dot(acc_e, wme_ref[...], preferred_element_type=jnp.float32)
              ) * invd
        hn = hn + jnp.where(invd > 0, 1.0, 0.0) * bm_ref[...]
        z = (jnp.dot(h_self, was_ref[...], preferred_element_type=jnp.float32)
             + jnp.dot(hn, wan_ref[...], preferred_element_type=jnp.float32)
             + ba_ref[...])
        return jnp.maximum(z, 0.0)

    def mega_kernel(src_smem, eid_smem, dst_smem, dst_vmem, h0_ref, ef_ref,
                    wmn0, wme0, bm0, was0, wan0, ba0,
                    wmn1, wme1, bm1, was1, wan1, ba1,
                    out_ref, slabh_ref, slabe_ref,
                    acch_ref, acce_ref, accd_ref, h1_ref):
        t = pl.program_id(0)

        @pl.when(t == 0)
        def _():
            acch_ref[...] = jnp.zeros_like(acch_ref)
            acce_ref[...] = jnp.zeros_like(acce_ref)
            accd_ref[...] = jnp.zeros_like(accd_ref)

        @pl.when(t < NTILES)
        def _():
            agg_tile(src_smem, eid_smem, dst_smem, dst_vmem, h0_ref, ef_ref,
                     slabh_ref, slabe_ref, acch_ref, acce_ref, accd_ref)

        @pl.when(jnp.logical_and(t >= NTILES, t < P1))
        def _():
            b = t - NTILES
            sl = pl.ds(b * TN, TN)
            cnt = accd_ref[sl, :]
            invd = jnp.where(cnt > 0, 1.0 / cnt, 0.0)
            h1_ref[sl, :] = apply_block(acch_ref[sl, :], acce_ref[sl, :],
                                        h0_ref[sl, :], invd,
                                        wmn0, wme0, bm0, was0, wan0, ba0)

        @pl.when(t == P1)
        def _():
            acch_ref[...] = jnp.zeros_like(acch_ref)

        @pl.when(jnp.logical_and(t >= P1, t < P2))
        def _():
            agg_tile(src_smem, eid_smem, dst_smem, dst_vmem, h1_ref, None,
                     slabh_ref, None, acch_ref, None, None)

        @pl.when(t >= P2)
        def _():
            b = t - P2
            sl = pl.ds(b * TN, TN)
            cnt = accd_ref[sl, :]
            invd = jnp.where(cnt > 0, 1.0 / cnt, 0.0)
            out_ref[...] = apply_block(acch_ref[sl, :], acce_ref[sl, :],
                                       h1_ref[sl, :], invd,
                                       wmn1, wme1, bm1, was1, wan1, ba1)

    def tile_map(t):
        u = jnp.where(t < P1, t, t - P1)
        return (0, jnp.clip(u, 0, NTILES - 1))

    def out_map(t):
        return (jnp.maximum(t - P2, 0), 0)

    rspec = lambda shape: pl.BlockSpec(shape, lambda t: (0, 0))
    wspecs = [rspec((LANE, LANE)), rspec((LANE, LANE)), rspec((1, LANE)),
              rspec((LANE, LANE)), rspec((LANE, LANE)), rspec((1, LANE))]

    out1 = pl.pallas_call(
        mega_kernel,
        out_shape=jax.ShapeDtypeStruct((N, LANE), jnp.float32),
        grid_spec=pltpu.PrefetchScalarGridSpec(
            num_scalar_prefetch=0,
            grid=(GRID,),
            in_specs=[
                pl.BlockSpec((1, TE), tile_map, memory_space=pltpu.SMEM),
                pl.BlockSpec((1, TE), tile_map, memory_space=pltpu.SMEM),
                pl.BlockSpec((1, TE), tile_map, memory_space=pltpu.SMEM),
                pl.BlockSpec((1, TE), tile_map),
                rspec((N, LANE)),
                rspec((E, LANE)),
                *wspecs, *wspecs,
            ],
            out_specs=pl.BlockSpec((TN, LANE), out_map),
            scratch_shapes=[pltpu.VMEM((TE, LANE), jnp.float32),
                            pltpu.VMEM((TE, LANE), jnp.float32),
                            pltpu.VMEM((N, LANE), jnp.float32),
                            pltpu.VMEM((N, LANE), jnp.float32),
                            pltpu.VMEM((N, 1), jnp.float32),
                            pltpu.VMEM((N, LANE), jnp.float32)],
        ),
        compiler_params=pltpu.CompilerParams(
            dimension_semantics=("arbitrary",),
            vmem_limit_bytes=VMEM_LIMIT,
        ),
    )(srcs, eid, dsts, dsts, h0, ef,
      l0_Wm_n, l0_Wm_e, l0_b_msg, l0_Wa_s, l0_Wa_n, l0_b_apply,
      l1_Wm_n, l1_Wm_e, l1_b_msg, l1_Wa_s, l1_Wa_n, l1_b_apply)

    return out1
